# baseline (speedup 1.0000x reference)
import functools

import jax
import jax.numpy as jnp
from jax import lax
from jax.experimental import pallas as pl
from jax.experimental.pallas import tpu as pltpu
from jax.experimental.pallas import tpu_sc as plsc

_N = 10000
_E = 320000
_D = 128

_NC = 2
_NS = 16
_NPAD = 10240
_ROWS_PER_TILE = _NPAD // _NS
_ZROWS = 32


def _vec_mesh():
    return plsc.VectorSubcoreMesh(
        core_axis_name="c", subcore_axis_name="s",
        num_cores=_NC, num_subcores=_NS)


def _fill_const(ref, nrows, value):
    vec = jnp.full((16,), value, jnp.float32)

    def row(i, carry):
        def lane(j, carry2):
            ref[i, pl.ds(j * 16, 16)] = vec
            return carry2
        return lax.fori_loop(0, _D // 16, lane, carry)
    lax.fori_loop(0, nrows, row, 0)


def _zero_acc_slice(zbuf, acc_sh, s):
    def za(i, carry):
        pltpu.sync_copy(
            zbuf, acc_sh.at[pl.ds(s * _ROWS_PER_TILE + i * _ZROWS, _ZROWS)])
        return carry
    lax.fori_loop(0, _ROWS_PER_TILE // _ZROWS, za, 0)


_DEG_K = 128
_DEG_SEC = 327680
_DEG_EPT = _DEG_SEC // _NS
_DEG_ITERS = _DEG_EPT // _DEG_K


def _sc_degrees_body(sd_hbm, out_hbm, ones_v, idx0, idx1, zbuf, acc_sh,
                     sem0, sem1):
    c = lax.axis_index("c")
    s = lax.axis_index("s")

    _fill_const(ones_v, _DEG_K, 1.0)
    _fill_const(zbuf, _ZROWS, 0.0)
    _zero_acc_slice(zbuf, acc_sh, s)
    plsc.subcore_barrier()

    base = c * _DEG_SEC + s * _DEG_EPT

    def _fire(j, ibuf, sem):
        pltpu.async_copy(sd_hbm.at[pl.ds(base + j * _DEG_K, _DEG_K)],
                         ibuf, sem)

    def _drain(j, ibuf, sem):
        pltpu.make_async_copy(sd_hbm.at[pl.ds(base + j * _DEG_K, _DEG_K)],
                              ibuf, sem).wait()

    _fire(0, idx0, sem0)
    _fire(1, idx1, sem1)

    def step(p, carry):
        j = 2 * p
        _drain(j, idx0, sem0)
        pltpu.sync_copy(ones_v, acc_sh.at[idx0], add=True)
        _fire(j + 2, idx0, sem0)
        _drain(j + 1, idx1, sem1)
        pltpu.sync_copy(ones_v, acc_sh.at[idx1], add=True)
        _fire(j + 3, idx1, sem1)
        return carry
    lax.fori_loop(0, _DEG_ITERS // 2, step, 0)
    _drain(_DEG_ITERS, idx0, sem0)
    _drain(_DEG_ITERS + 1, idx1, sem1)
    plsc.subcore_barrier()

    r0 = s * _ROWS_PER_TILE
    pltpu.sync_copy(acc_sh.at[pl.ds(r0, _ROWS_PER_TILE)],
                    out_hbm.at[c, pl.ds(r0, _ROWS_PER_TILE)])


def _sc_degrees(sd):
    k = pl.kernel(
        _sc_degrees_body,
        out_type=jax.ShapeDtypeStruct((2, _NPAD, _D), jnp.float32),
        mesh=_vec_mesh(),
        scratch_types=[
            pltpu.VMEM((_DEG_K, _D), jnp.float32),
            pltpu.VMEM((_DEG_K,), jnp.int32),
            pltpu.VMEM((_DEG_K,), jnp.int32),
            pltpu.VMEM((_ZROWS, _D), jnp.float32),
            pltpu.VMEM_SHARED((_NPAD, _D), jnp.float32),
            pltpu.SemaphoreType.DMA,
            pltpu.SemaphoreType.DMA,
        ],
    )
    return k(sd)


_SP_K = 128
_SP_E2 = 327680
_SP_EPT = _SP_E2 // (_NC * _NS)
_SP_ITERS = _SP_EPT // _SP_K


def _sc_spmm_body(xs_hbm, src_hbm, dst_hbm, out_hbm,
                  sidxa, didxa, rowsa, isema, gsema,
                  sidxb, didxb, rowsb, isemb, gsemb,
                  zbuf, acc_sh):
    c = lax.axis_index("c")
    s = lax.axis_index("s")

    _fill_const(zbuf, _ZROWS, 0.0)
    _zero_acc_slice(zbuf, acc_sh, s)
    plsc.subcore_barrier()

    base = c * (_SP_E2 // _NC) + s * _SP_EPT

    def fire_idx(j, sidx, didx, isem):
        off = base + j * _SP_K
        pltpu.async_copy(src_hbm.at[pl.ds(off, _SP_K)], sidx, isem)
        pltpu.async_copy(dst_hbm.at[pl.ds(off, _SP_K)], didx, isem)

    def drain_idx(j, sidx, didx, isem):
        off = base + j * _SP_K
        pltpu.make_async_copy(src_hbm.at[pl.ds(off, _SP_K)], sidx, isem).wait()
        pltpu.make_async_copy(dst_hbm.at[pl.ds(off, _SP_K)], didx, isem).wait()

    def fire_g(sidx, rows, gsem):
        pltpu.async_copy(xs_hbm.at[sidx], rows, gsem)

    def drain_g(sidx, rows, gsem):
        pltpu.make_async_copy(xs_hbm.at[sidx], rows, gsem).wait()

    def scat(rows, didx):
        pltpu.sync_copy(rows, acc_sh.at[didx], add=True)

    fire_idx(0, sidxa, didxa, isema)
    fire_idx(1, sidxb, didxb, isemb)
    drain_idx(0, sidxa, didxa, isema)
    fire_g(sidxa, rowsa, gsema)

    def step(p, carry):
        j = 2 * p
        drain_idx(j + 1, sidxb, didxb, isemb)
        fire_g(sidxb, rowsb, gsemb)
        drain_g(sidxa, rowsa, gsema)
        scat(rowsa, didxa)
        fire_idx(j + 2, sidxa, didxa, isema)
        drain_idx(j + 2, sidxa, didxa, isema)
        fire_g(sidxa, rowsa, gsema)
        drain_g(sidxb, rowsb, gsemb)
        scat(rowsb, didxb)
        fire_idx(j + 3, sidxb, didxb, isemb)
        return carry
    lax.fori_loop(0, _SP_ITERS // 2, step, 0)
    drain_g(sidxa, rowsa, gsema)
    scat(rowsa, didxa)
    drain_idx(_SP_ITERS + 1, sidxb, didxb, isemb)
    plsc.subcore_barrier()

    r0 = s * _ROWS_PER_TILE
    pltpu.sync_copy(acc_sh.at[pl.ds(r0, _ROWS_PER_TILE)],
                    out_hbm.at[c, pl.ds(r0, _ROWS_PER_TILE)])


def _sc_spmm(xs, src, dst):
    k = pl.kernel(
        _sc_spmm_body,
        out_type=jax.ShapeDtypeStruct((2, _NPAD, _D), jnp.float32),
        mesh=_vec_mesh(),
        scratch_types=[
            pltpu.VMEM((_SP_K,), jnp.int32),
            pltpu.VMEM((_SP_K,), jnp.int32),
            pltpu.VMEM((_SP_K, _D), jnp.float32),
            pltpu.SemaphoreType.DMA,
            pltpu.SemaphoreType.DMA,
            pltpu.VMEM((_SP_K,), jnp.int32),
            pltpu.VMEM((_SP_K,), jnp.int32),
            pltpu.VMEM((_SP_K, _D), jnp.float32),
            pltpu.SemaphoreType.DMA,
            pltpu.SemaphoreType.DMA,
            pltpu.VMEM((_ZROWS, _D), jnp.float32),
            pltpu.VMEM_SHARED((_NPAD, _D), jnp.float32),
        ],
    )
    return k(xs, src, dst)


_R = 1000


def _norms_from_deg(deg_blk):
    dout = deg_blk[0, :, 0:1]
    din = deg_blk[1, :, 0:1]
    ns = lax.rsqrt(jnp.where(dout > 0, dout, 1.0))
    nd = lax.rsqrt(jnp.where(din > 0, din, 1.0))
    return ns, nd


def _tc_prep_body(x_ref, deg_ref, o_ref):
    ns, _ = _norms_from_deg(deg_ref[...])
    o_ref[...] = x_ref[...] * ns


def _tc_prep(x, deg):
    return pl.pallas_call(
        _tc_prep_body,
        grid=(_N // _R,),
        in_specs=[
            pl.BlockSpec((_R, _D), lambda i: (i, 0)),
            pl.BlockSpec((2, _R, _D), lambda i: (0, i, 0)),
        ],
        out_specs=pl.BlockSpec((_R, _D), lambda i: (i, 0)),
        out_shape=jax.ShapeDtypeStruct((_N, _D), jnp.float32),
    )(x, deg)


def _tc_layer_body(final, g_ref, deg_ref, w_ref, b_ref, ga_ref, be_ref,
                   x_ref, o_ref):
    ns, nd = _norms_from_deg(deg_ref[...])
    gsum = (g_ref[0] + g_ref[1]) * nd
    h = jnp.dot(gsum, w_ref[...], preferred_element_type=jnp.float32)
    h = h + b_ref[...]
    m = jnp.mean(h, axis=-1, keepdims=True)
    v = jnp.mean((h - m) ** 2, axis=-1, keepdims=True)
    h = (h - m) * lax.rsqrt(v + 1e-5) * ga_ref[...] + be_ref[...]
    h = h * jax.nn.sigmoid(h)
    if final:
        o_ref[...] = h + x_ref[...]
    else:
        o_ref[...] = h * ns


def _tc_layer(gpart, deg, w, b, ga, be, x, final):
    return pl.pallas_call(
        functools.partial(_tc_layer_body, final),
        grid=(_N // _R,),
        in_specs=[
            pl.BlockSpec((2, _R, _D), lambda i: (0, i, 0)),
            pl.BlockSpec((2, _R, _D), lambda i: (0, i, 0)),
            pl.BlockSpec((_D, _D), lambda i: (0, 0)),
            pl.BlockSpec((1, _D), lambda i: (0, 0)),
            pl.BlockSpec((1, _D), lambda i: (0, 0)),
            pl.BlockSpec((1, _D), lambda i: (0, 0)),
            pl.BlockSpec((_R, _D), lambda i: (i, 0)),
        ],
        out_specs=pl.BlockSpec((_R, _D), lambda i: (i, 0)),
        out_shape=jax.ShapeDtypeStruct((_N, _D), jnp.float32),
    )(gpart, deg, w, b, ga, be, x)


def kernel(x, edge_index, W1, b1, g1, be1, W2, b2, g2, be2):
    ei = edge_index.astype(jnp.int32)
    src = ei[0]
    dst = ei[1]
    b1r, g1r, be1r = b1.reshape(1, _D), g1.reshape(1, _D), be1.reshape(1, _D)
    b2r, g2r, be2r = b2.reshape(1, _D), g2.reshape(1, _D), be2.reshape(1, _D)

    sink = _NPAD - 1
    np_sp = _SP_E2 - _E + 2 * _SP_K
    srcp = jnp.pad(src, (0, np_sp))
    dstp = jnp.concatenate([dst, jnp.full((np_sp,), sink, jnp.int32)])
    dpad = jnp.full((_DEG_SEC - _E,), sink, jnp.int32)
    sd = jnp.concatenate([src, dpad, dst, dpad,
                          jnp.zeros((2 * _DEG_K,), jnp.int32)])
    deg = _sc_degrees(sd)[:, :_N]
    t0 = _tc_prep(x, deg)
    gp1 = _sc_spmm(t0, srcp, dstp)[:, :_N]
    t1 = _tc_layer(gp1, deg, W1, b1r, g1r, be1r, x, final=False)
    gp2 = _sc_spmm(t1, srcp, dstp)[:, :_N]
    out = _tc_layer(gp2, deg, W2, b2r, g2r, be2r, x, final=True)
    return out


# speedup vs baseline: 1.0443x; 1.0443x over previous
import functools

import jax
import jax.numpy as jnp
from jax import lax
from jax.experimental import pallas as pl
from jax.experimental.pallas import tpu as pltpu
from jax.experimental.pallas import tpu_sc as plsc

_N = 10000
_E = 320000
_D = 128

_NC = 2
_NS = 16
_NPAD = 10240
_ROWS_PER_TILE = _NPAD // _NS
_ZROWS = 32


def _vec_mesh():
    return plsc.VectorSubcoreMesh(
        core_axis_name="c", subcore_axis_name="s",
        num_cores=_NC, num_subcores=_NS)


def _fill_const(ref, nrows, value):
    vec = jnp.full((16,), value, jnp.float32)

    def row(i, carry):
        def lane(j, carry2):
            ref[i, pl.ds(j * 16, 16)] = vec
            return carry2
        return lax.fori_loop(0, _D // 16, lane, carry)
    lax.fori_loop(0, nrows, row, 0)


def _zero_acc_slice(zbuf, acc_sh, s):
    def za(i, carry):
        pltpu.sync_copy(
            zbuf, acc_sh.at[pl.ds(s * _ROWS_PER_TILE + i * _ZROWS, _ZROWS)])
        return carry
    lax.fori_loop(0, _ROWS_PER_TILE // _ZROWS, za, 0)


_DEG_K = 80
_DEG_ITERS = 256
_DEG_EPT = _DEG_ITERS * _DEG_K
_DEG_SEC = _DEG_EPT * _NS


def _sc_degrees_body(sd_hbm, out_hbm, ones_v, idxf, zbuf, acc_sh):
    c = lax.axis_index("c")
    s = lax.axis_index("s")

    _fill_const(ones_v, _DEG_K, 1.0)
    _fill_const(zbuf, _ZROWS, 0.0)
    _zero_acc_slice(zbuf, acc_sh, s)

    base = c * _DEG_SEC + s * _DEG_EPT
    pltpu.sync_copy(sd_hbm.at[pl.ds(base, _DEG_EPT)], idxf)
    plsc.subcore_barrier()

    def step(j, carry):
        pltpu.sync_copy(ones_v,
                        acc_sh.at[idxf.at[pl.ds(j * _DEG_K, _DEG_K)]],
                        add=True)
        return carry
    lax.fori_loop(0, _DEG_ITERS, step, 0)
    plsc.subcore_barrier()

    r0 = s * _ROWS_PER_TILE
    pltpu.sync_copy(acc_sh.at[pl.ds(r0, _ROWS_PER_TILE)],
                    out_hbm.at[c, pl.ds(r0, _ROWS_PER_TILE)])


def _sc_degrees(sd):
    k = pl.kernel(
        _sc_degrees_body,
        out_type=jax.ShapeDtypeStruct((2, _NPAD, _D), jnp.float32),
        mesh=_vec_mesh(),
        scratch_types=[
            pltpu.VMEM((_DEG_K, _D), jnp.float32),
            pltpu.VMEM((_DEG_EPT,), jnp.int32),
            pltpu.VMEM((_ZROWS, _D), jnp.float32),
            pltpu.VMEM_SHARED((_NPAD, _D), jnp.float32),
        ],
    )
    return k(sd)


_SP_K = 80
_SP_ITERS = 128
_SP_EPT = _SP_ITERS * _SP_K
_SP_E2 = _SP_EPT * _NC * _NS


def _sc_spmm_body(xs_hbm, src_hbm, dst_hbm, out_hbm,
                  sidxf, didxf, rowsa, gsema, rowsb, gsemb,
                  zbuf, acc_sh):
    c = lax.axis_index("c")
    s = lax.axis_index("s")

    _fill_const(zbuf, _ZROWS, 0.0)
    _zero_acc_slice(zbuf, acc_sh, s)

    base = c * (_SP_E2 // _NC) + s * _SP_EPT
    pltpu.sync_copy(src_hbm.at[pl.ds(base, _SP_EPT)], sidxf)
    pltpu.sync_copy(dst_hbm.at[pl.ds(base, _SP_EPT)], didxf)
    plsc.subcore_barrier()

    def _six(j):
        return sidxf.at[pl.ds(j * _SP_K, _SP_K)]

    def fire_g(j, rows, gsem):
        pltpu.async_copy(xs_hbm.at[_six(j)], rows, gsem)

    def drain_g(j, rows, gsem):
        pltpu.make_async_copy(xs_hbm.at[_six(j)], rows, gsem).wait()

    def scat(j, rows):
        pltpu.sync_copy(rows,
                        acc_sh.at[didxf.at[pl.ds(j * _SP_K, _SP_K)]],
                        add=True)

    fire_g(0, rowsa, gsema)

    def step(p, carry):
        j = 2 * p
        fire_g(j + 1, rowsb, gsemb)
        drain_g(j, rowsa, gsema)
        scat(j, rowsa)

        @pl.when(j + 2 < _SP_ITERS)
        def _():
            fire_g(j + 2, rowsa, gsema)
        drain_g(j + 1, rowsb, gsemb)
        scat(j + 1, rowsb)
        return carry
    lax.fori_loop(0, _SP_ITERS // 2, step, 0)
    plsc.subcore_barrier()

    r0 = s * _ROWS_PER_TILE
    pltpu.sync_copy(acc_sh.at[pl.ds(r0, _ROWS_PER_TILE)],
                    out_hbm.at[c, pl.ds(r0, _ROWS_PER_TILE)])


def _sc_spmm(xs, src1, dst1):
    k = pl.kernel(
        _sc_spmm_body,
        out_type=jax.ShapeDtypeStruct((2, _NPAD, _D), jnp.float32),
        mesh=_vec_mesh(),
        scratch_types=[
            pltpu.VMEM((_SP_EPT,), jnp.int32),
            pltpu.VMEM((_SP_EPT,), jnp.int32),
            pltpu.VMEM((_SP_K, _D), jnp.float32),
            pltpu.SemaphoreType.DMA,
            pltpu.VMEM((_SP_K, _D), jnp.float32),
            pltpu.SemaphoreType.DMA,
            pltpu.VMEM((_ZROWS, _D), jnp.float32),
            pltpu.VMEM_SHARED((_NPAD, _D), jnp.float32),
        ],
    )
    return k(xs, src1, dst1)


_R = 1000


def _norms_from_deg(deg_blk):
    dout = deg_blk[0, :, 0:1]
    din = deg_blk[1, :, 0:1]
    ns = lax.rsqrt(jnp.where(dout > 0, dout, 1.0))
    nd = lax.rsqrt(jnp.where(din > 0, din, 1.0))
    return ns, nd


def _tc_prep_body(x_ref, deg_ref, o_ref):
    ns, _ = _norms_from_deg(deg_ref[...])
    o_ref[...] = x_ref[...] * ns


def _tc_prep(x, deg):
    return pl.pallas_call(
        _tc_prep_body,
        grid=(_N // _R,),
        in_specs=[
            pl.BlockSpec((_R, _D), lambda i: (i, 0)),
            pl.BlockSpec((2, _R, _D), lambda i: (0, i, 0)),
        ],
        out_specs=pl.BlockSpec((_R, _D), lambda i: (i, 0)),
        out_shape=jax.ShapeDtypeStruct((_N, _D), jnp.float32),
    )(x, deg)


def _tc_layer_body(final, g_ref, deg_ref, w_ref, b_ref, ga_ref, be_ref,
                   x_ref, o_ref):
    ns, nd = _norms_from_deg(deg_ref[...])
    gsum = (g_ref[0] + g_ref[1]) * nd
    h = jnp.dot(gsum, w_ref[...], preferred_element_type=jnp.float32)
    h = h + b_ref[...]
    m = jnp.mean(h, axis=-1, keepdims=True)
    v = jnp.mean((h - m) ** 2, axis=-1, keepdims=True)
    h = (h - m) * lax.rsqrt(v + 1e-5) * ga_ref[...] + be_ref[...]
    h = h * jax.nn.sigmoid(h)
    if final:
        o_ref[...] = h + x_ref[...]
    else:
        o_ref[...] = h * ns


def _tc_layer(gpart, deg, w, b, ga, be, x, final):
    return pl.pallas_call(
        functools.partial(_tc_layer_body, final),
        grid=(_N // _R,),
        in_specs=[
            pl.BlockSpec((2, _R, _D), lambda i: (0, i, 0)),
            pl.BlockSpec((2, _R, _D), lambda i: (0, i, 0)),
            pl.BlockSpec((_D, _D), lambda i: (0, 0)),
            pl.BlockSpec((1, _D), lambda i: (0, 0)),
            pl.BlockSpec((1, _D), lambda i: (0, 0)),
            pl.BlockSpec((1, _D), lambda i: (0, 0)),
            pl.BlockSpec((_R, _D), lambda i: (i, 0)),
        ],
        out_specs=pl.BlockSpec((_R, _D), lambda i: (i, 0)),
        out_shape=jax.ShapeDtypeStruct((_N, _D), jnp.float32),
    )(gpart, deg, w, b, ga, be, x)


def kernel(x, edge_index, W1, b1, g1, be1, W2, b2, g2, be2):
    ei = edge_index.astype(jnp.int32)
    src = ei[0]
    dst = ei[1]
    b1r, g1r, be1r = b1.reshape(1, _D), g1.reshape(1, _D), be1.reshape(1, _D)
    b2r, g2r, be2r = b2.reshape(1, _D), g2.reshape(1, _D), be2.reshape(1, _D)

    sink = _NPAD - 1
    srcp = jnp.pad(src, (0, _SP_E2 - _E))
    dstp = jnp.concatenate([dst, jnp.full((_SP_E2 - _E,), sink, jnp.int32)])
    dpad = jnp.full((_DEG_SEC - _E,), sink, jnp.int32)
    sd = jnp.concatenate([src, dpad, dst, dpad])
    deg = _sc_degrees(sd)[:, :_N]
    t0 = _tc_prep(x, deg)
    gp1 = _sc_spmm(t0, srcp, dstp)[:, :_N]
    t1 = _tc_layer(gp1, deg, W1, b1r, g1r, be1r, x, final=False)
    gp2 = _sc_spmm(t1, srcp, dstp)[:, :_N]
    out = _tc_layer(gp2, deg, W2, b2r, g2r, be2r, x, final=True)
    return out


# speedup vs baseline: 1.0608x; 1.0159x over previous
import functools

import jax
import jax.numpy as jnp
from jax import lax
from jax.experimental import pallas as pl
from jax.experimental.pallas import tpu as pltpu
from jax.experimental.pallas import tpu_sc as plsc

_N = 10000
_E = 320000
_D = 128

_NC = 2
_NS = 16
_NPAD = 10240
_ROWS_PER_TILE = _NPAD // _NS
_ZROWS = 32


def _vec_mesh():
    return plsc.VectorSubcoreMesh(
        core_axis_name="c", subcore_axis_name="s",
        num_cores=_NC, num_subcores=_NS)


def _fill_const(ref, nrows, value):
    vec = jnp.full((16,), value, jnp.float32)

    def row(i, carry):
        def lane(j, carry2):
            ref[i, pl.ds(j * 16, 16)] = vec
            return carry2
        return lax.fori_loop(0, _D // 16, lane, carry)
    lax.fori_loop(0, nrows, row, 0)


def _zero_acc_slice(zbuf, acc_sh, s):
    def za(i, carry):
        pltpu.sync_copy(
            zbuf, acc_sh.at[pl.ds(s * _ROWS_PER_TILE + i * _ZROWS, _ZROWS)])
        return carry
    lax.fori_loop(0, _ROWS_PER_TILE // _ZROWS, za, 0)


_DEG_K = 80
_DEG_ITERS = 256
_DEG_EPT = _DEG_ITERS * _DEG_K
_DEG_SEC = _DEG_EPT * _NS


def _sc_degrees_body(sd_hbm, out_hbm, ones_v, idxf, zbuf, acc_sh):
    c = lax.axis_index("c")
    s = lax.axis_index("s")

    _fill_const(ones_v, _DEG_K, 1.0)
    _fill_const(zbuf, _ZROWS, 0.0)
    _zero_acc_slice(zbuf, acc_sh, s)

    base = c * _DEG_SEC + s * _DEG_EPT
    pltpu.sync_copy(sd_hbm.at[pl.ds(base, _DEG_EPT)], idxf)
    plsc.subcore_barrier()

    def step(j, carry):
        pltpu.sync_copy(ones_v,
                        acc_sh.at[idxf.at[pl.ds(j * _DEG_K, _DEG_K)]],
                        add=True)
        return carry
    lax.fori_loop(0, _DEG_ITERS, step, 0)
    plsc.subcore_barrier()

    r0 = s * _ROWS_PER_TILE
    pltpu.sync_copy(acc_sh.at[pl.ds(r0, _ROWS_PER_TILE)],
                    out_hbm.at[c, pl.ds(r0, _ROWS_PER_TILE)])


def _sc_degrees(sd):
    k = pl.kernel(
        _sc_degrees_body,
        out_type=jax.ShapeDtypeStruct((2, _NPAD, _D), jnp.float32),
        mesh=_vec_mesh(),
        scratch_types=[
            pltpu.VMEM((_DEG_K, _D), jnp.float32),
            pltpu.VMEM((_DEG_EPT,), jnp.int32),
            pltpu.VMEM((_ZROWS, _D), jnp.float32),
            pltpu.VMEM_SHARED((_NPAD, _D), jnp.float32),
        ],
    )
    return k(sd)


_SP_K = 80
_SP_ITERS = 128
_SP_EPT = _SP_ITERS * _SP_K
_SP_E2 = _SP_EPT * _NC * _NS


def _sc_spmm_body(xs_hbm, src_hbm, dst_hbm, out_hbm,
                  sidxf, didxf, rowsa, gsema, rowsb, gsemb,
                  zbuf, acc_sh):
    c = lax.axis_index("c")
    s = lax.axis_index("s")

    _fill_const(zbuf, _ZROWS, 0.0)
    _zero_acc_slice(zbuf, acc_sh, s)

    base = c * (_SP_E2 // _NC) + s * _SP_EPT
    pltpu.sync_copy(src_hbm.at[pl.ds(base, _SP_EPT)], sidxf)
    pltpu.sync_copy(dst_hbm.at[pl.ds(base, _SP_EPT)], didxf)
    plsc.subcore_barrier()

    def _six(j):
        return sidxf.at[pl.ds(j * _SP_K, _SP_K)]

    def fire_g(j, rows, gsem):
        pltpu.async_copy(xs_hbm.at[_six(j)], rows, gsem)

    def drain_g(j, rows, gsem):
        pltpu.make_async_copy(xs_hbm.at[_six(j)], rows, gsem).wait()

    def scat(j, rows):
        pltpu.sync_copy(rows,
                        acc_sh.at[didxf.at[pl.ds(j * _SP_K, _SP_K)]],
                        add=True)

    fire_g(0, rowsa, gsema)

    def step(p, carry):
        j = 2 * p
        fire_g(j + 1, rowsb, gsemb)
        drain_g(j, rowsa, gsema)
        scat(j, rowsa)

        @pl.when(j + 2 < _SP_ITERS)
        def _():
            fire_g(j + 2, rowsa, gsema)
        drain_g(j + 1, rowsb, gsemb)
        scat(j + 1, rowsb)
        return carry
    lax.fori_loop(0, _SP_ITERS // 2, step, 0)
    plsc.subcore_barrier()

    r0 = s * _ROWS_PER_TILE
    pltpu.sync_copy(acc_sh.at[pl.ds(r0, _ROWS_PER_TILE)],
                    out_hbm.at[c, pl.ds(r0, _ROWS_PER_TILE)])


def _sc_spmm(xs, src1, dst1):
    k = pl.kernel(
        _sc_spmm_body,
        out_type=jax.ShapeDtypeStruct((2, _NPAD, _D), jnp.float32),
        mesh=_vec_mesh(),
        scratch_types=[
            pltpu.VMEM((_SP_EPT,), jnp.int32),
            pltpu.VMEM((_SP_EPT,), jnp.int32),
            pltpu.VMEM((_SP_K, _D), jnp.float32),
            pltpu.SemaphoreType.DMA,
            pltpu.VMEM((_SP_K, _D), jnp.float32),
            pltpu.SemaphoreType.DMA,
            pltpu.VMEM((_ZROWS, _D), jnp.float32),
            pltpu.VMEM_SHARED((_NPAD, _D), jnp.float32),
        ],
    )
    return k(xs, src1, dst1)


_R = 1000


def _norms_from_deg(deg_blk):
    dout = deg_blk[0, :, 0:1]
    din = deg_blk[1, :, 0:1]
    ns = lax.rsqrt(jnp.where(dout > 0, dout, 1.0))
    nd = lax.rsqrt(jnp.where(din > 0, din, 1.0))
    return ns, nd


def _tc_prep_body(x_ref, deg_ref, o_ref):
    ns, _ = _norms_from_deg(deg_ref[...])
    o_ref[...] = x_ref[...] * ns


def _tc_prep(x, deg):
    return pl.pallas_call(
        _tc_prep_body,
        grid=(_N // _R,),
        in_specs=[
            pl.BlockSpec((_R, _D), lambda i: (i, 0)),
            pl.BlockSpec((2, _R, _D), lambda i: (0, i, 0)),
        ],
        out_specs=pl.BlockSpec((_R, _D), lambda i: (i, 0)),
        out_shape=jax.ShapeDtypeStruct((_N, _D), jnp.float32),
    )(x, deg)


def _tc_layer_body(final, g_ref, deg_ref, w_ref, b_ref, ga_ref, be_ref,
                   x_ref, o_ref):
    ns, nd = _norms_from_deg(deg_ref[...])
    gsum = (g_ref[0] + g_ref[1]) * nd
    h = jnp.dot(gsum, w_ref[...], preferred_element_type=jnp.float32)
    h = h + b_ref[...]
    m = jnp.mean(h, axis=-1, keepdims=True)
    v = jnp.mean((h - m) ** 2, axis=-1, keepdims=True)
    h = (h - m) * lax.rsqrt(v + 1e-5) * ga_ref[...] + be_ref[...]
    h = h * jax.nn.sigmoid(h)
    if final:
        o_ref[...] = h + x_ref[...]
    else:
        o_ref[...] = h * ns


def _tc_layer(gpart, deg, w, b, ga, be, x, final):
    return pl.pallas_call(
        functools.partial(_tc_layer_body, final),
        grid=(_N // _R,),
        in_specs=[
            pl.BlockSpec((2, _R, _D), lambda i: (0, i, 0)),
            pl.BlockSpec((2, _R, _D), lambda i: (0, i, 0)),
            pl.BlockSpec((_D, _D), lambda i: (0, 0)),
            pl.BlockSpec((1, _D), lambda i: (0, 0)),
            pl.BlockSpec((1, _D), lambda i: (0, 0)),
            pl.BlockSpec((1, _D), lambda i: (0, 0)),
            pl.BlockSpec((_R, _D), lambda i: (i, 0)),
        ],
        out_specs=pl.BlockSpec((_R, _D), lambda i: (i, 0)),
        out_shape=jax.ShapeDtypeStruct((_N, _D), jnp.float32),
    )(gpart, deg, w, b, ga, be, x)


def kernel(x, edge_index, W1, b1, g1, be1, W2, b2, g2, be2):
    ei = edge_index.astype(jnp.int32)
    src = ei[0]
    dst = ei[1]
    b1r, g1r, be1r = b1.reshape(1, _D), g1.reshape(1, _D), be1.reshape(1, _D)
    b2r, g2r, be2r = b2.reshape(1, _D), g2.reshape(1, _D), be2.reshape(1, _D)

    npad_sp = _SP_E2 - _E
    sinks = _N + (jnp.arange(npad_sp, dtype=jnp.int32) % (_NPAD - _N))
    srcp = jnp.pad(src, (0, npad_sp))
    dstp = jnp.concatenate([dst, sinks])
    dpad = _N + (jnp.arange(_DEG_SEC - _E, dtype=jnp.int32) % (_NPAD - _N))
    sd = jnp.concatenate([src, dpad, dst, dpad])
    deg = _sc_degrees(sd)[:, :_N]
    t0 = _tc_prep(x, deg)
    gp1 = _sc_spmm(t0, srcp, dstp)[:, :_N]
    t1 = _tc_layer(gp1, deg, W1, b1r, g1r, be1r, x, final=False)
    gp2 = _sc_spmm(t1, srcp, dstp)[:, :_N]
    out = _tc_layer(gp2, deg, W2, b2r, g2r, be2r, x, final=True)
    return out


# speedup vs baseline: 2.6751x; 2.5217x over previous
import functools

import jax
import jax.numpy as jnp
from jax import lax
from jax.experimental import pallas as pl
from jax.experimental.pallas import tpu as pltpu
from jax.experimental.pallas import tpu_sc as plsc

_N = 10000
_E = 320000
_D = 128

_NC = 2
_NS = 16
_NPAD = 10240
_ROWS_PER_TILE = _NPAD // _NS
_ZROWS = 32


def _vec_mesh():
    return plsc.VectorSubcoreMesh(
        core_axis_name="c", subcore_axis_name="s",
        num_cores=_NC, num_subcores=_NS)


def _fill_const(ref, nrows, value):
    vec = jnp.full((16,), value, jnp.float32)

    def row(i, carry):
        def lane(j, carry2):
            ref[i, pl.ds(j * 16, 16)] = vec
            return carry2
        return lax.fori_loop(0, _D // 16, lane, carry)
    lax.fori_loop(0, nrows, row, 0)


def _zero_acc_slice(zbuf, acc_sh, s):
    def za(i, carry):
        pltpu.sync_copy(
            zbuf, acc_sh.at[pl.ds(s * _ROWS_PER_TILE + i * _ZROWS, _ZROWS)])
        return carry
    lax.fori_loop(0, _ROWS_PER_TILE // _ZROWS, za, 0)


_DEG_K = 80
_DEG_ITERS = 256
_DEG_EPT = _DEG_ITERS * _DEG_K
_DEG_SEC = _DEG_EPT * _NS


def _sc_degrees_body(sd_hbm, out_hbm, ones_v, idxf, zbuf, acc_sh):
    c = lax.axis_index("c")
    s = lax.axis_index("s")

    _fill_const(ones_v, _DEG_K, 1.0)
    _fill_const(zbuf, _ZROWS, 0.0)
    _zero_acc_slice(zbuf, acc_sh, s)

    base = c * _DEG_SEC + s * _DEG_EPT
    pltpu.sync_copy(sd_hbm.at[pl.ds(base, _DEG_EPT)], idxf)
    plsc.subcore_barrier()

    def step(j, carry):
        pltpu.sync_copy(ones_v,
                        acc_sh.at[idxf.at[pl.ds(j * _DEG_K, _DEG_K)]],
                        add=True)
        return carry
    lax.fori_loop(0, _DEG_ITERS, step, 0)
    plsc.subcore_barrier()

    r0 = s * _ROWS_PER_TILE
    pltpu.sync_copy(acc_sh.at[pl.ds(r0, _ROWS_PER_TILE)],
                    out_hbm.at[c, pl.ds(r0, _ROWS_PER_TILE)])


def _sc_degrees(sd):
    k = pl.kernel(
        _sc_degrees_body,
        out_type=jax.ShapeDtypeStruct((2, _NPAD, _D), jnp.float32),
        mesh=_vec_mesh(),
        scratch_types=[
            pltpu.VMEM((_DEG_K, _D), jnp.float32),
            pltpu.VMEM((_DEG_EPT,), jnp.int32),
            pltpu.VMEM((_ZROWS, _D), jnp.float32),
            pltpu.VMEM_SHARED((_NPAD, _D), jnp.float32),
        ],
    )
    return k(sd)


_SP_K = 80
_SP_ITERS = 128
_SP_EPT = _SP_ITERS * _SP_K
_SP_E2 = _SP_EPT * _NC * _NS


def _sc_spmm_body(xs_hbm, src_hbm, dst_hbm, out_hbm,
                  sidxf, didxf, rowsa, gsema, rowsb, gsemb,
                  zbuf, acc_sh):
    c = lax.axis_index("c")
    s = lax.axis_index("s")

    _fill_const(zbuf, _ZROWS, 0.0)
    _zero_acc_slice(zbuf, acc_sh, s)

    base = c * (_SP_E2 // _NC) + s * _SP_EPT
    pltpu.sync_copy(src_hbm.at[pl.ds(base, _SP_EPT)], sidxf)
    pltpu.sync_copy(dst_hbm.at[pl.ds(base, _SP_EPT)], didxf)
    plsc.subcore_barrier()

    def _six(j):
        return sidxf.at[pl.ds(j * _SP_K, _SP_K)]

    def fire_g(j, rows, gsem):
        pltpu.async_copy(xs_hbm.at[_six(j)], rows, gsem)

    def drain_g(j, rows, gsem):
        pltpu.make_async_copy(xs_hbm.at[_six(j)], rows, gsem).wait()

    def scat(j, rows):
        pltpu.sync_copy(rows,
                        acc_sh.at[didxf.at[pl.ds(j * _SP_K, _SP_K)]],
                        add=True)

    fire_g(0, rowsa, gsema)

    def step(p, carry):
        j = 2 * p
        fire_g(j + 1, rowsb, gsemb)
        drain_g(j, rowsa, gsema)
        scat(j, rowsa)

        @pl.when(j + 2 < _SP_ITERS)
        def _():
            fire_g(j + 2, rowsa, gsema)
        drain_g(j + 1, rowsb, gsemb)
        scat(j + 1, rowsb)
        return carry
    lax.fori_loop(0, _SP_ITERS // 2, step, 0)
    plsc.subcore_barrier()

    r0 = s * _ROWS_PER_TILE
    pltpu.sync_copy(acc_sh.at[pl.ds(r0, _ROWS_PER_TILE)],
                    out_hbm.at[c, pl.ds(r0, _ROWS_PER_TILE)])


def _sc_spmm(xs, src1, dst1):
    k = pl.kernel(
        _sc_spmm_body,
        out_type=jax.ShapeDtypeStruct((2, _NPAD, _D), jnp.float32),
        mesh=_vec_mesh(),
        scratch_types=[
            pltpu.VMEM((_SP_EPT,), jnp.int32),
            pltpu.VMEM((_SP_EPT,), jnp.int32),
            pltpu.VMEM((_SP_K, _D), jnp.float32),
            pltpu.SemaphoreType.DMA,
            pltpu.VMEM((_SP_K, _D), jnp.float32),
            pltpu.SemaphoreType.DMA,
            pltpu.VMEM((_ZROWS, _D), jnp.float32),
            pltpu.VMEM_SHARED((_NPAD, _D), jnp.float32),
        ],
    )
    return k(xs, src1, dst1)


_R = 1000


def _norms_from_deg(deg_blk):
    dout = deg_blk[0, :, 0:1]
    din = deg_blk[1, :, 0:1]
    ns = lax.rsqrt(jnp.where(dout > 0, dout, 1.0))
    nd = lax.rsqrt(jnp.where(din > 0, din, 1.0))
    return ns, nd


def _tc_prep_body(x_ref, deg_ref, o_ref):
    ns, _ = _norms_from_deg(deg_ref[...])
    o_ref[...] = x_ref[...] * ns


def _tc_prep(x, deg):
    return pl.pallas_call(
        _tc_prep_body,
        grid=(_N // _R,),
        in_specs=[
            pl.BlockSpec((_R, _D), lambda i: (i, 0)),
            pl.BlockSpec((2, _R, _D), lambda i: (0, i, 0)),
        ],
        out_specs=pl.BlockSpec((_R, _D), lambda i: (i, 0)),
        out_shape=jax.ShapeDtypeStruct((_N, _D), jnp.float32),
    )(x, deg)


def _tc_layer_body(final, g_ref, deg_ref, w_ref, b_ref, ga_ref, be_ref,
                   x_ref, o_ref):
    ns, nd = _norms_from_deg(deg_ref[...])
    gsum = (g_ref[0] + g_ref[1]) * nd
    h = jnp.dot(gsum, w_ref[...], preferred_element_type=jnp.float32)
    h = h + b_ref[...]
    m = jnp.mean(h, axis=-1, keepdims=True)
    v = jnp.mean((h - m) ** 2, axis=-1, keepdims=True)
    h = (h - m) * lax.rsqrt(v + 1e-5) * ga_ref[...] + be_ref[...]
    h = h * jax.nn.sigmoid(h)
    if final:
        o_ref[...] = h + x_ref[...]
    else:
        o_ref[...] = h * ns


def _tc_layer(gpart, deg, w, b, ga, be, x, final):
    return pl.pallas_call(
        functools.partial(_tc_layer_body, final),
        grid=(_N // _R,),
        in_specs=[
            pl.BlockSpec((2, _R, _D), lambda i: (0, i, 0)),
            pl.BlockSpec((2, _R, _D), lambda i: (0, i, 0)),
            pl.BlockSpec((_D, _D), lambda i: (0, 0)),
            pl.BlockSpec((1, _D), lambda i: (0, 0)),
            pl.BlockSpec((1, _D), lambda i: (0, 0)),
            pl.BlockSpec((1, _D), lambda i: (0, 0)),
            pl.BlockSpec((_R, _D), lambda i: (i, 0)),
        ],
        out_specs=pl.BlockSpec((_R, _D), lambda i: (i, 0)),
        out_shape=jax.ShapeDtypeStruct((_N, _D), jnp.float32),
    )(gpart, deg, w, b, ga, be, x)


def kernel(x, edge_index, W1, b1, g1, be1, W2, b2, g2, be2):
    ei = edge_index.astype(jnp.int32)
    src = ei[0]
    dst = ei[1]
    b1r, g1r, be1r = b1.reshape(1, _D), g1.reshape(1, _D), be1.reshape(1, _D)
    b2r, g2r, be2r = b2.reshape(1, _D), g2.reshape(1, _D), be2.reshape(1, _D)

    npad_sp = _SP_E2 - _E
    sinks = _N + (jnp.arange(npad_sp, dtype=jnp.int32) % (_NPAD - _N))
    spread = jnp.arange(npad_sp, dtype=jnp.int32) % _N
    srcp = jnp.concatenate([src, spread])
    dstp = jnp.concatenate([dst, sinks])
    dpad = _N + (jnp.arange(_DEG_SEC - _E, dtype=jnp.int32) % (_NPAD - _N))
    sd = jnp.concatenate([src, dpad, dst, dpad])
    deg = _sc_degrees(sd)[:, :_N]
    t0 = _tc_prep(x, deg)
    gp1 = _sc_spmm(t0, srcp, dstp)[:, :_N]
    t1 = _tc_layer(gp1, deg, W1, b1r, g1r, be1r, x, final=False)
    gp2 = _sc_spmm(t1, srcp, dstp)[:, :_N]
    out = _tc_layer(gp2, deg, W2, b2r, g2r, be2r, x, final=True)
    return out


# speedup vs baseline: 2.7212x; 1.0172x over previous
import functools

import jax
import jax.numpy as jnp
from jax import lax
from jax.experimental import pallas as pl
from jax.experimental.pallas import tpu as pltpu
from jax.experimental.pallas import tpu_sc as plsc

_N = 10000
_E = 320000
_D = 128

_NC = 2
_NS = 16
_NPAD = 10240
_ROWS_PER_TILE = _NPAD // _NS
_ZROWS = 32


def _vec_mesh():
    return plsc.VectorSubcoreMesh(
        core_axis_name="c", subcore_axis_name="s",
        num_cores=_NC, num_subcores=_NS)


def _fill_const(ref, nrows, value):
    vec = jnp.full((16,), value, jnp.float32)

    def row(i, carry):
        def lane(j, carry2):
            ref[i, pl.ds(j * 16, 16)] = vec
            return carry2
        return lax.fori_loop(0, _D // 16, lane, carry)
    lax.fori_loop(0, nrows, row, 0)


def _zero_acc_slice(zbuf, acc_sh, s):
    def za(i, carry):
        pltpu.sync_copy(
            zbuf, acc_sh.at[pl.ds(s * _ROWS_PER_TILE + i * _ZROWS, _ZROWS)])
        return carry
    lax.fori_loop(0, _ROWS_PER_TILE // _ZROWS, za, 0)


_DEG_K = 80
_DEG_EPT = _E // _NS
_DEG_ITERS = _DEG_EPT // _DEG_K


def _sc_degrees_body(src_hbm, dst_hbm, out_hbm, ones_v, idxf, zbuf, acc_sh):
    c = lax.axis_index("c")
    s = lax.axis_index("s")

    _fill_const(ones_v, _DEG_K, 1.0)
    _fill_const(zbuf, _ZROWS, 0.0)
    _zero_acc_slice(zbuf, acc_sh, s)

    @pl.when(c == 0)
    def _():
        pltpu.sync_copy(src_hbm.at[pl.ds(s * _DEG_EPT, _DEG_EPT)], idxf)

    @pl.when(c == 1)
    def _():
        pltpu.sync_copy(dst_hbm.at[pl.ds(s * _DEG_EPT, _DEG_EPT)], idxf)
    plsc.subcore_barrier()

    def step(j, carry):
        pltpu.sync_copy(ones_v,
                        acc_sh.at[idxf.at[pl.ds(j * _DEG_K, _DEG_K)]],
                        add=True)
        return carry
    lax.fori_loop(0, _DEG_ITERS, step, 0)
    plsc.subcore_barrier()

    r0 = s * _ROWS_PER_TILE
    pltpu.sync_copy(acc_sh.at[pl.ds(r0, _ROWS_PER_TILE)],
                    out_hbm.at[c, pl.ds(r0, _ROWS_PER_TILE)])


def _sc_degrees(src, dst):
    k = pl.kernel(
        _sc_degrees_body,
        out_type=jax.ShapeDtypeStruct((2, _NPAD, _D), jnp.float32),
        mesh=_vec_mesh(),
        scratch_types=[
            pltpu.VMEM((_DEG_K, _D), jnp.float32),
            pltpu.VMEM((_DEG_EPT,), jnp.int32),
            pltpu.VMEM((_ZROWS, _D), jnp.float32),
            pltpu.VMEM_SHARED((_NPAD, _D), jnp.float32),
        ],
    )
    return k(src, dst)


_SP_K = 80
_SP_EPT = _E // (_NC * _NS)
_SP_ITERS = _SP_EPT // _SP_K


def _sc_spmm_body(xs_hbm, src_hbm, dst_hbm, out_hbm,
                  sidxf, didxf, rowsa, gsema, rowsb, gsemb,
                  zbuf, acc_sh):
    c = lax.axis_index("c")
    s = lax.axis_index("s")

    _fill_const(zbuf, _ZROWS, 0.0)
    _zero_acc_slice(zbuf, acc_sh, s)

    base = c * (_E // _NC) + s * _SP_EPT
    pltpu.sync_copy(src_hbm.at[pl.ds(base, _SP_EPT)], sidxf)
    pltpu.sync_copy(dst_hbm.at[pl.ds(base, _SP_EPT)], didxf)
    plsc.subcore_barrier()

    def _six(j):
        return sidxf.at[pl.ds(j * _SP_K, _SP_K)]

    def fire_g(j, rows, gsem):
        pltpu.async_copy(xs_hbm.at[_six(j)], rows, gsem)

    def drain_g(j, rows, gsem):
        pltpu.make_async_copy(xs_hbm.at[_six(j)], rows, gsem).wait()

    def scat(j, rows):
        pltpu.sync_copy(rows,
                        acc_sh.at[didxf.at[pl.ds(j * _SP_K, _SP_K)]],
                        add=True)

    fire_g(0, rowsa, gsema)

    def step(p, carry):
        j = 2 * p
        fire_g(j + 1, rowsb, gsemb)
        drain_g(j, rowsa, gsema)
        scat(j, rowsa)
        fire_g(j + 2, rowsa, gsema)
        drain_g(j + 1, rowsb, gsemb)
        scat(j + 1, rowsb)
        return carry
    lax.fori_loop(0, _SP_ITERS // 2, step, 0)
    drain_g(_SP_ITERS - 1, rowsa, gsema)
    scat(_SP_ITERS - 1, rowsa)
    plsc.subcore_barrier()

    r0 = s * _ROWS_PER_TILE
    pltpu.sync_copy(acc_sh.at[pl.ds(r0, _ROWS_PER_TILE)],
                    out_hbm.at[c, pl.ds(r0, _ROWS_PER_TILE)])


def _sc_spmm(xs, src1, dst1):
    k = pl.kernel(
        _sc_spmm_body,
        out_type=jax.ShapeDtypeStruct((2, _NPAD, _D), jnp.float32),
        mesh=_vec_mesh(),
        scratch_types=[
            pltpu.VMEM((_SP_EPT,), jnp.int32),
            pltpu.VMEM((_SP_EPT,), jnp.int32),
            pltpu.VMEM((_SP_K, _D), jnp.float32),
            pltpu.SemaphoreType.DMA,
            pltpu.VMEM((_SP_K, _D), jnp.float32),
            pltpu.SemaphoreType.DMA,
            pltpu.VMEM((_ZROWS, _D), jnp.float32),
            pltpu.VMEM_SHARED((_NPAD, _D), jnp.float32),
        ],
    )
    return k(xs, src1, dst1)


_R = 1000


def _norms_from_deg(deg_blk):
    dout = deg_blk[0, :, 0:1]
    din = deg_blk[1, :, 0:1]
    ns = lax.rsqrt(jnp.where(dout > 0, dout, 1.0))
    nd = lax.rsqrt(jnp.where(din > 0, din, 1.0))
    return ns, nd


def _tc_prep_body(x_ref, deg_ref, o_ref):
    ns, _ = _norms_from_deg(deg_ref[...])
    o_ref[...] = x_ref[...] * ns


def _tc_prep(x, deg):
    return pl.pallas_call(
        _tc_prep_body,
        grid=(_N // _R,),
        in_specs=[
            pl.BlockSpec((_R, _D), lambda i: (i, 0)),
            pl.BlockSpec((2, _R, _D), lambda i: (0, i, 0)),
        ],
        out_specs=pl.BlockSpec((_R, _D), lambda i: (i, 0)),
        out_shape=jax.ShapeDtypeStruct((_N, _D), jnp.float32),
    )(x, deg)


def _tc_layer_body(final, g_ref, deg_ref, w_ref, b_ref, ga_ref, be_ref,
                   x_ref, o_ref):
    ns, nd = _norms_from_deg(deg_ref[...])
    gsum = (g_ref[0] + g_ref[1]) * nd
    h = jnp.dot(gsum, w_ref[...], preferred_element_type=jnp.float32)
    h = h + b_ref[...]
    m = jnp.mean(h, axis=-1, keepdims=True)
    v = jnp.mean((h - m) ** 2, axis=-1, keepdims=True)
    h = (h - m) * lax.rsqrt(v + 1e-5) * ga_ref[...] + be_ref[...]
    h = h * jax.nn.sigmoid(h)
    if final:
        o_ref[...] = h + x_ref[...]
    else:
        o_ref[...] = h * ns


def _tc_layer(gpart, deg, w, b, ga, be, x, final):
    return pl.pallas_call(
        functools.partial(_tc_layer_body, final),
        grid=(_N // _R,),
        in_specs=[
            pl.BlockSpec((2, _R, _D), lambda i: (0, i, 0)),
            pl.BlockSpec((2, _R, _D), lambda i: (0, i, 0)),
            pl.BlockSpec((_D, _D), lambda i: (0, 0)),
            pl.BlockSpec((1, _D), lambda i: (0, 0)),
            pl.BlockSpec((1, _D), lambda i: (0, 0)),
            pl.BlockSpec((1, _D), lambda i: (0, 0)),
            pl.BlockSpec((_R, _D), lambda i: (i, 0)),
        ],
        out_specs=pl.BlockSpec((_R, _D), lambda i: (i, 0)),
        out_shape=jax.ShapeDtypeStruct((_N, _D), jnp.float32),
    )(gpart, deg, w, b, ga, be, x)


def kernel(x, edge_index, W1, b1, g1, be1, W2, b2, g2, be2):
    ei = edge_index.astype(jnp.int32)
    src = ei[0]
    dst = ei[1]
    b1r, g1r, be1r = b1.reshape(1, _D), g1.reshape(1, _D), be1.reshape(1, _D)
    b2r, g2r, be2r = b2.reshape(1, _D), g2.reshape(1, _D), be2.reshape(1, _D)

    deg = _sc_degrees(src, dst)[:, :_N]
    t0 = _tc_prep(x, deg)
    gp1 = _sc_spmm(t0, src, dst)[:, :_N]
    t1 = _tc_layer(gp1, deg, W1, b1r, g1r, be1r, x, final=False)
    gp2 = _sc_spmm(t1, src, dst)[:, :_N]
    out = _tc_layer(gp2, deg, W2, b2r, g2r, be2r, x, final=True)
    return out


# speedup vs baseline: 2.8571x; 1.0500x over previous
import functools

import jax
import jax.numpy as jnp
from jax import lax
from jax.experimental import pallas as pl
from jax.experimental.pallas import tpu as pltpu
from jax.experimental.pallas import tpu_sc as plsc

_N = 10000
_E = 320000
_D = 128

_NC = 2
_NS = 16
_NPAD = 10240
_ROWS_PER_TILE = _NPAD // _NS
_ZROWS = 32


def _vec_mesh():
    return plsc.VectorSubcoreMesh(
        core_axis_name="c", subcore_axis_name="s",
        num_cores=_NC, num_subcores=_NS)


def _fill_const(ref, nrows, value):
    vec = jnp.full((16,), value, jnp.float32)

    def row(i, carry):
        def lane(j, carry2):
            ref[i, pl.ds(j * 16, 16)] = vec
            return carry2
        return lax.fori_loop(0, _D // 16, lane, carry)
    lax.fori_loop(0, nrows, row, 0)


def _zero_acc_slice(zbuf, acc_sh, s):
    def za(i, carry):
        pltpu.sync_copy(
            zbuf, acc_sh.at[pl.ds(s * _ROWS_PER_TILE + i * _ZROWS, _ZROWS)])
        return carry
    lax.fori_loop(0, _ROWS_PER_TILE // _ZROWS, za, 0)


_DEG_K = 80
_DEG_EPT = _E // _NS
_DEG_ITERS = _DEG_EPT // _DEG_K


def _sc_degrees_body(src_hbm, dst_hbm, out_hbm, ones_v, idxf, zbuf, acc_sh):
    c = lax.axis_index("c")
    s = lax.axis_index("s")

    _fill_const(ones_v, _DEG_K, 1.0)
    _fill_const(zbuf, _ZROWS, 0.0)
    _zero_acc_slice(zbuf, acc_sh, s)

    @pl.when(c == 0)
    def _():
        pltpu.sync_copy(src_hbm.at[pl.ds(s * _DEG_EPT, _DEG_EPT)], idxf)

    @pl.when(c == 1)
    def _():
        pltpu.sync_copy(dst_hbm.at[pl.ds(s * _DEG_EPT, _DEG_EPT)], idxf)
    plsc.subcore_barrier()

    def step(j, carry):
        pltpu.sync_copy(ones_v,
                        acc_sh.at[idxf.at[pl.ds(j * _DEG_K, _DEG_K)]],
                        add=True)
        return carry
    lax.fori_loop(0, _DEG_ITERS, step, 0)
    plsc.subcore_barrier()

    r0 = s * _ROWS_PER_TILE
    pltpu.sync_copy(acc_sh.at[pl.ds(r0, _ROWS_PER_TILE)],
                    out_hbm.at[c, pl.ds(r0, _ROWS_PER_TILE)])


def _sc_degrees(src, dst):
    k = pl.kernel(
        _sc_degrees_body,
        out_type=jax.ShapeDtypeStruct((2, _NPAD, _D), jnp.float32),
        mesh=_vec_mesh(),
        scratch_types=[
            pltpu.VMEM((_DEG_K, _D), jnp.float32),
            pltpu.VMEM((_DEG_EPT,), jnp.int32),
            pltpu.VMEM((_ZROWS, _D), jnp.float32),
            pltpu.VMEM_SHARED((_NPAD, _D), jnp.float32),
        ],
    )
    return k(src, dst)


_SP_K = 80
_SP_EPT = _E // (_NC * _NS)
_SP_ITERS = _SP_EPT // _SP_K


def _sc_spmm_body(xs_hbm, src_hbm, dst_hbm, out_hbm,
                  sidxf, didxf, rowsa, gsema, rowsb, gsemb,
                  zbuf, acc_sh):
    c = lax.axis_index("c")
    s = lax.axis_index("s")

    _fill_const(zbuf, _ZROWS, 0.0)
    _zero_acc_slice(zbuf, acc_sh, s)

    base = c * (_E // _NC) + s * _SP_EPT
    pltpu.sync_copy(src_hbm.at[pl.ds(base, _SP_EPT)], sidxf)
    pltpu.sync_copy(dst_hbm.at[pl.ds(base, _SP_EPT)], didxf)
    plsc.subcore_barrier()

    def _six(j):
        return sidxf.at[pl.ds(j * _SP_K, _SP_K)]

    def fire_g(j, rows, gsem):
        pltpu.async_copy(xs_hbm.at[_six(j)], rows, gsem)

    def drain_g(j, rows, gsem):
        pltpu.make_async_copy(xs_hbm.at[_six(j)], rows, gsem).wait()

    def scat(j, rows):
        pltpu.sync_copy(rows,
                        acc_sh.at[didxf.at[pl.ds(j * _SP_K, _SP_K)]],
                        add=True)

    fire_g(0, rowsa, gsema)

    def step(p, carry):
        j = 2 * p
        fire_g(j + 1, rowsb, gsemb)
        drain_g(j, rowsa, gsema)
        scat(j, rowsa)
        fire_g(j + 2, rowsa, gsema)
        drain_g(j + 1, rowsb, gsemb)
        scat(j + 1, rowsb)
        return carry
    lax.fori_loop(0, _SP_ITERS // 2, step, 0)
    drain_g(_SP_ITERS - 1, rowsa, gsema)
    scat(_SP_ITERS - 1, rowsa)
    plsc.subcore_barrier()

    r0 = s * _ROWS_PER_TILE
    pltpu.sync_copy(acc_sh.at[pl.ds(r0, _ROWS_PER_TILE)],
                    out_hbm.at[c, pl.ds(r0, _ROWS_PER_TILE)])


def _sc_spmm(xs, src1, dst1):
    k = pl.kernel(
        _sc_spmm_body,
        out_type=jax.ShapeDtypeStruct((2, _NPAD, _D), jnp.float32),
        mesh=_vec_mesh(),
        scratch_types=[
            pltpu.VMEM((_SP_EPT,), jnp.int32),
            pltpu.VMEM((_SP_EPT,), jnp.int32),
            pltpu.VMEM((_SP_K, _D), jnp.float32),
            pltpu.SemaphoreType.DMA,
            pltpu.VMEM((_SP_K, _D), jnp.float32),
            pltpu.SemaphoreType.DMA,
            pltpu.VMEM((_ZROWS, _D), jnp.float32),
            pltpu.VMEM_SHARED((_NPAD, _D), jnp.float32),
        ],
    )
    return k(xs, src1, dst1)


_R = 1000


def _norms_from_deg(deg_blk):
    dout = deg_blk[0, :, 0:1]
    din = deg_blk[1, :, 0:1]
    ns = lax.rsqrt(jnp.where(dout > 0, dout, 1.0))
    nd = lax.rsqrt(jnp.where(din > 0, din, 1.0))
    return ns, nd


def _tc_prep_body(x_ref, deg_ref, o_ref):
    ns, _ = _norms_from_deg(deg_ref[...])
    o_ref[...] = x_ref[...] * ns


def _tc_prep(x, deg):
    return pl.pallas_call(
        _tc_prep_body,
        grid=(_N // _R,),
        in_specs=[
            pl.BlockSpec((_R, _D), lambda i: (i, 0)),
            pl.BlockSpec((2, _R, _D), lambda i: (0, i, 0)),
        ],
        out_specs=pl.BlockSpec((_R, _D), lambda i: (i, 0)),
        out_shape=jax.ShapeDtypeStruct((_N, _D), jnp.float32),
    )(x, deg)


def _tc_layer_body(final, g_ref, deg_ref, w_ref, b_ref, ga_ref, be_ref,
                   x_ref, o_ref):
    ns, nd = _norms_from_deg(deg_ref[...])
    gsum = (g_ref[0] + g_ref[1]) * nd
    h = jnp.dot(gsum, w_ref[...], preferred_element_type=jnp.float32)
    h = h + b_ref[...]
    m = jnp.mean(h, axis=-1, keepdims=True)
    v = jnp.mean((h - m) ** 2, axis=-1, keepdims=True)
    h = (h - m) * lax.rsqrt(v + 1e-5) * ga_ref[...] + be_ref[...]
    h = h * jax.nn.sigmoid(h)
    if final:
        o_ref[...] = h + x_ref[...]
    else:
        o_ref[...] = h * ns


def _tc_layer(gpart, deg, w, b, ga, be, x, final):
    return pl.pallas_call(
        functools.partial(_tc_layer_body, final),
        grid=(_N // _R,),
        in_specs=[
            pl.BlockSpec((2, _R, _D), lambda i: (0, i, 0)),
            pl.BlockSpec((2, _R, _D), lambda i: (0, i, 0)),
            pl.BlockSpec((_D, _D), lambda i: (0, 0)),
            pl.BlockSpec((1, _D), lambda i: (0, 0)),
            pl.BlockSpec((1, _D), lambda i: (0, 0)),
            pl.BlockSpec((1, _D), lambda i: (0, 0)),
            pl.BlockSpec((_R, _D), lambda i: (i, 0)),
        ],
        out_specs=pl.BlockSpec((_R, _D), lambda i: (i, 0)),
        out_shape=jax.ShapeDtypeStruct((_N, _D), jnp.float32),
    )(gpart, deg, w, b, ga, be, x)


def kernel(x, edge_index, W1, b1, g1, be1, W2, b2, g2, be2):
    ei = edge_index.astype(jnp.int32)
    src = ei[0]
    dst = ei[1]
    b1r, g1r, be1r = b1.reshape(1, _D), g1.reshape(1, _D), be1.reshape(1, _D)
    b2r, g2r, be2r = b2.reshape(1, _D), g2.reshape(1, _D), be2.reshape(1, _D)

    deg = _sc_degrees(src, dst)
    t0 = _tc_prep(x, deg)
    gp1 = _sc_spmm(t0, src, dst)
    t1 = _tc_layer(gp1, deg, W1, b1r, g1r, be1r, x, final=False)
    gp2 = _sc_spmm(t1, src, dst)
    out = _tc_layer(gp2, deg, W2, b2r, g2r, be2r, x, final=True)
    return out


# speedup vs baseline: 2.8884x; 1.0110x over previous
import functools

import jax
import jax.numpy as jnp
from jax import lax
from jax.experimental import pallas as pl
from jax.experimental.pallas import tpu as pltpu
from jax.experimental.pallas import tpu_sc as plsc

_N = 10000
_E = 320000
_D = 128

_NC = 2
_NS = 16
_NPAD = 10240
_ROWS_PER_TILE = _NPAD // _NS
_ZROWS = 32


def _vec_mesh():
    return plsc.VectorSubcoreMesh(
        core_axis_name="c", subcore_axis_name="s",
        num_cores=_NC, num_subcores=_NS)


def _fill_const(ref, nrows, value):
    vec = jnp.full((16,), value, jnp.float32)

    def row(i, carry):
        def lane(j, carry2):
            ref[i, pl.ds(j * 16, 16)] = vec
            return carry2
        return lax.fori_loop(0, _D // 16, lane, carry)
    lax.fori_loop(0, nrows, row, 0)


def _zero_acc_slice(zbuf, acc_sh, s):
    def za(i, carry):
        pltpu.sync_copy(
            zbuf, acc_sh.at[pl.ds(s * _ROWS_PER_TILE + i * _ZROWS, _ZROWS)])
        return carry
    lax.fori_loop(0, _ROWS_PER_TILE // _ZROWS, za, 0)


_DEG_K = 80
_DEG_EPT = _E // _NS
_DEG_ITERS = _DEG_EPT // _DEG_K


def _sc_degrees_body(src_hbm, dst_hbm, out_hbm, ones_v, idxf, zbuf, acc_sh):
    c = lax.axis_index("c")
    s = lax.axis_index("s")

    _fill_const(ones_v, _DEG_K, 1.0)
    _fill_const(zbuf, _ZROWS, 0.0)
    _zero_acc_slice(zbuf, acc_sh, s)

    @pl.when(c == 0)
    def _():
        pltpu.sync_copy(src_hbm.at[pl.ds(s * _DEG_EPT, _DEG_EPT)], idxf)

    @pl.when(c == 1)
    def _():
        pltpu.sync_copy(dst_hbm.at[pl.ds(s * _DEG_EPT, _DEG_EPT)], idxf)
    plsc.subcore_barrier()

    def step(j, carry):
        pltpu.sync_copy(ones_v,
                        acc_sh.at[idxf.at[pl.ds(j * _DEG_K, _DEG_K)]],
                        add=True)
        return carry
    lax.fori_loop(0, _DEG_ITERS, step, 0)
    plsc.subcore_barrier()

    r0 = s * _ROWS_PER_TILE
    pltpu.sync_copy(acc_sh.at[pl.ds(r0, _ROWS_PER_TILE)],
                    out_hbm.at[c, pl.ds(r0, _ROWS_PER_TILE)])


def _sc_degrees(src, dst):
    k = pl.kernel(
        _sc_degrees_body,
        out_type=jax.ShapeDtypeStruct((2, _NPAD, _D), jnp.float32),
        mesh=_vec_mesh(),
        scratch_types=[
            pltpu.VMEM((_DEG_K, _D), jnp.float32),
            pltpu.VMEM((_DEG_EPT,), jnp.int32),
            pltpu.VMEM((_ZROWS, _D), jnp.float32),
            pltpu.VMEM_SHARED((_NPAD, _D), jnp.float32),
        ],
    )
    return k(src, dst)


_SP_K = 80
_SP_EPT = _E // (_NC * _NS)
_SP_ITERS = _SP_EPT // _SP_K


def _sc_spmm_body(xs_hbm, src_hbm, dst_hbm, out_hbm,
                  sidxf, didxf, rowsa, gsema, rowsb, gsemb,
                  zbuf, acc_sh):
    c = lax.axis_index("c")
    s = lax.axis_index("s")

    _fill_const(zbuf, _ZROWS, 0.0)
    _zero_acc_slice(zbuf, acc_sh, s)

    base = c * (_E // _NC) + s * _SP_EPT
    pltpu.sync_copy(src_hbm.at[pl.ds(base, _SP_EPT)], sidxf)
    pltpu.sync_copy(dst_hbm.at[pl.ds(base, _SP_EPT)], didxf)
    plsc.subcore_barrier()

    def _six(j):
        return sidxf.at[pl.ds(j * _SP_K, _SP_K)]

    def fire_g(j, rows, gsem):
        pltpu.async_copy(xs_hbm.at[_six(j)], rows, gsem)

    def drain_g(j, rows, gsem):
        pltpu.make_async_copy(xs_hbm.at[_six(j)], rows, gsem).wait()

    def scat(j, rows):
        pltpu.sync_copy(rows,
                        acc_sh.at[didxf.at[pl.ds(j * _SP_K, _SP_K)]],
                        add=True)

    fire_g(0, rowsa, gsema)

    def step(p, carry):
        j = 2 * p
        fire_g(j + 1, rowsb, gsemb)
        drain_g(j, rowsa, gsema)
        scat(j, rowsa)
        fire_g(j + 2, rowsa, gsema)
        drain_g(j + 1, rowsb, gsemb)
        scat(j + 1, rowsb)
        return carry
    lax.fori_loop(0, _SP_ITERS // 2, step, 0)
    drain_g(_SP_ITERS - 1, rowsa, gsema)
    scat(_SP_ITERS - 1, rowsa)
    plsc.subcore_barrier()

    r0 = s * _ROWS_PER_TILE
    pltpu.sync_copy(acc_sh.at[pl.ds(r0, _ROWS_PER_TILE)],
                    out_hbm.at[c, pl.ds(r0, _ROWS_PER_TILE)])


def _sc_spmm(xs, src1, dst1):
    k = pl.kernel(
        _sc_spmm_body,
        out_type=jax.ShapeDtypeStruct((2, _NPAD, _D), jnp.float32),
        mesh=_vec_mesh(),
        scratch_types=[
            pltpu.VMEM((_SP_EPT,), jnp.int32),
            pltpu.VMEM((_SP_EPT,), jnp.int32),
            pltpu.VMEM((_SP_K, _D), jnp.float32),
            pltpu.SemaphoreType.DMA,
            pltpu.VMEM((_SP_K, _D), jnp.float32),
            pltpu.SemaphoreType.DMA,
            pltpu.VMEM((_ZROWS, _D), jnp.float32),
            pltpu.VMEM_SHARED((_NPAD, _D), jnp.float32),
        ],
    )
    return k(xs, src1, dst1)


_R = 2000


def _norms_from_deg(deg_blk):
    dout = deg_blk[0, :, 0:1]
    din = deg_blk[1, :, 0:1]
    ns = lax.rsqrt(jnp.where(dout > 0, dout, 1.0))
    nd = lax.rsqrt(jnp.where(din > 0, din, 1.0))
    return ns, nd


def _tc_prep_body(x_ref, deg_ref, o_ref):
    ns, _ = _norms_from_deg(deg_ref[...])
    o_ref[...] = x_ref[...] * ns


def _tc_prep(x, deg):
    return pl.pallas_call(
        _tc_prep_body,
        grid=(_N // _R,),
        in_specs=[
            pl.BlockSpec((_R, _D), lambda i: (i, 0)),
            pl.BlockSpec((2, _R, _D), lambda i: (0, i, 0)),
        ],
        out_specs=pl.BlockSpec((_R, _D), lambda i: (i, 0)),
        out_shape=jax.ShapeDtypeStruct((_N, _D), jnp.float32),
    )(x, deg)


def _tc_layer_body(final, g_ref, deg_ref, w_ref, b_ref, ga_ref, be_ref,
                   x_ref, o_ref):
    ns, nd = _norms_from_deg(deg_ref[...])
    gsum = (g_ref[0] + g_ref[1]) * nd
    h = jnp.dot(gsum, w_ref[...], preferred_element_type=jnp.float32)
    h = h + b_ref[...]
    m = jnp.mean(h, axis=-1, keepdims=True)
    v = jnp.mean((h - m) ** 2, axis=-1, keepdims=True)
    h = (h - m) * lax.rsqrt(v + 1e-5) * ga_ref[...] + be_ref[...]
    h = h * jax.nn.sigmoid(h)
    if final:
        o_ref[...] = h + x_ref[...]
    else:
        o_ref[...] = h * ns


def _tc_layer(gpart, deg, w, b, ga, be, x, final):
    return pl.pallas_call(
        functools.partial(_tc_layer_body, final),
        grid=(_N // _R,),
        in_specs=[
            pl.BlockSpec((2, _R, _D), lambda i: (0, i, 0)),
            pl.BlockSpec((2, _R, _D), lambda i: (0, i, 0)),
            pl.BlockSpec((_D, _D), lambda i: (0, 0)),
            pl.BlockSpec((1, _D), lambda i: (0, 0)),
            pl.BlockSpec((1, _D), lambda i: (0, 0)),
            pl.BlockSpec((1, _D), lambda i: (0, 0)),
            pl.BlockSpec((_R, _D), lambda i: (i, 0)),
        ],
        out_specs=pl.BlockSpec((_R, _D), lambda i: (i, 0)),
        out_shape=jax.ShapeDtypeStruct((_N, _D), jnp.float32),
    )(gpart, deg, w, b, ga, be, x)


def kernel(x, edge_index, W1, b1, g1, be1, W2, b2, g2, be2):
    ei = edge_index.astype(jnp.int32)
    src = ei[0]
    dst = ei[1]
    b1r, g1r, be1r = b1.reshape(1, _D), g1.reshape(1, _D), be1.reshape(1, _D)
    b2r, g2r, be2r = b2.reshape(1, _D), g2.reshape(1, _D), be2.reshape(1, _D)

    deg = _sc_degrees(src, dst)
    t0 = _tc_prep(x, deg)
    gp1 = _sc_spmm(t0, src, dst)
    t1 = _tc_layer(gp1, deg, W1, b1r, g1r, be1r, x, final=False)
    gp2 = _sc_spmm(t1, src, dst)
    out = _tc_layer(gp2, deg, W2, b2r, g2r, be2r, x, final=True)
    return out
